# baseline (speedup 1.0000x reference)
import functools

import jax
import jax.numpy as jnp
from jax import lax
from jax.experimental import pallas as pl
from jax.experimental.pallas import tpu as pltpu
from jax.experimental.pallas import tpu_sc as plsc

NC = 2
NS = 16
NW = NC * NS
K = 128
NB = 4
NBA = 4
KA = 64
CORE0_FRAC = 1.0
BR = 512
DEGW = 128


def _sc_degree(dst_pad, zerosW, onesW, NP, EPW, CH):
    mesh = plsc.VectorSubcoreMesh(
        core_axis_name="c", subcore_axis_name="s", num_cores=NC, num_subcores=NS)
    rpt = NP // NS
    NG = CH // NB

    @functools.partial(
        pl.kernel,
        out_type=jax.ShapeDtypeStruct((NC, NP, DEGW), jnp.float32),
        mesh=mesh,
        scratch_types=(
            [pltpu.VMEM((K,), jnp.int32) for _ in range(NB)]
            + [pltpu.VMEM((K, DEGW), jnp.float32)]
            + [pltpu.SemaphoreType.DMA for _ in range(2 * NB)]
            + [pltpu.VMEM_SHARED((NP, DEGW), jnp.float32)]
        ),
    )
    def k(dst_hbm, zeros_hbm, ones_hbm, out_hbm, *refs):
        didx = refs[0:NB]
        ones_v = refs[NB]
        isem = refs[NB + 1:2 * NB + 1]
        ssem = refs[2 * NB + 1:3 * NB + 1]
        acc_sh = refs[3 * NB + 1]
        c = lax.axis_index("c")
        s = lax.axis_index("s")
        wid = s * NC + c
        pltpu.sync_copy(zeros_hbm.at[pl.ds(s * rpt, rpt)],
                        acc_sh.at[pl.ds(s * rpt, rpt)])
        pltpu.sync_copy(ones_hbm, ones_v)
        plsc.subcore_barrier()
        e0 = pl.multiple_of(wid * EPW, 8)

        for b in range(NB):
            off = pl.multiple_of(e0 + b * K, 8)
            pltpu.async_copy(dst_hbm.at[pl.ds(off, K)], didx[b], isem[b])

        def group(j, carry):
            for b in range(NB):
                pltpu.make_async_copy(
                    dst_hbm.at[pl.ds(e0, K)], didx[b], isem[b]).wait()
                pltpu.async_copy(ones_v, acc_sh.at[didx[b]], ssem[b], add=True)

                @pl.when(j < NG - 1)
                def _():
                    off = pl.multiple_of(e0 + ((j + 1) * NB + b) * K, 8)
                    pltpu.make_async_copy(
                        ones_v, acc_sh.at[didx[b]], ssem[b]).wait()
                    pltpu.async_copy(dst_hbm.at[pl.ds(off, K)], didx[b], isem[b])
            return carry

        lax.fori_loop(0, NG, group, 0)
        for b in range(NB):
            pltpu.make_async_copy(ones_v, acc_sh.at[didx[b]], ssem[b]).wait()
        plsc.subcore_barrier()
        pltpu.sync_copy(acc_sh.at[pl.ds(s * rpt, rpt)],
                        out_hbm.at[c, pl.ds(s * rpt, rpt)])

    return k(dst_pad, zerosW, onesW)


def _sc_edge_aggregate(g, src_pad, dst_pad, zerosD, NP, D, EPW0, EPW1):
    mesh = plsc.VectorSubcoreMesh(
        core_axis_name="c", subcore_axis_name="s", num_cores=NC, num_subcores=NS)
    rpt = NP // NS
    NG0 = EPW0 // (NBA * KA)
    NG1 = EPW1 // (NBA * KA)

    @functools.partial(
        pl.kernel,
        out_type=jax.ShapeDtypeStruct((NC, NP, D), jnp.float32),
        mesh=mesh,
        scratch_types=(
            [pltpu.VMEM((KA,), jnp.int32) for _ in range(2 * NBA)]
            + [pltpu.VMEM((KA, D), jnp.float32) for _ in range(NBA)]
            + [pltpu.SemaphoreType.DMA for _ in range(4 * NBA)]
            + [pltpu.VMEM_SHARED((NP, D), jnp.float32)]
        ),
    )
    def k(g_hbm, src_hbm, dst_hbm, zeros_hbm, out_hbm, *refs):
        sidx = refs[0:NBA]
        didx = refs[NBA:2 * NBA]
        rows = refs[2 * NBA:3 * NBA]
        sisem = refs[3 * NBA:4 * NBA]
        disem = refs[4 * NBA:5 * NBA]
        gsem = refs[5 * NBA:6 * NBA]
        ssem = refs[6 * NBA:7 * NBA]
        acc_sh = refs[7 * NBA]
        c = lax.axis_index("c")
        s = lax.axis_index("s")
        wid = s * NC + c
        pltpu.sync_copy(zeros_hbm.at[pl.ds(s * rpt, rpt)],
                        acc_sh.at[pl.ds(s * rpt, rpt)])
        plsc.subcore_barrier()
        epw = jnp.where(c == 0, EPW0, EPW1)
        NG = jnp.where(c == 0, NG0, NG1)
        e0 = pl.multiple_of(c * (NS * EPW0) + s * epw, 8)

        for b in range(NBA):
            off = pl.multiple_of(e0 + b * KA, 8)
            pltpu.async_copy(src_hbm.at[pl.ds(off, KA)], sidx[b], sisem[b])
            pltpu.async_copy(dst_hbm.at[pl.ds(off, KA)], didx[b], disem[b])
            pltpu.make_async_copy(
                src_hbm.at[pl.ds(e0, KA)], sidx[b], sisem[b]).wait()
            pltpu.async_copy(g_hbm.at[sidx[b]], rows[b], gsem[b])

        def group(j, carry):
            for b in range(NBA):
                pltpu.make_async_copy(
                    g_hbm.at[sidx[b]], rows[b], gsem[b]).wait()

                @pl.when(j < NG - 1)
                def _():
                    off = pl.multiple_of(e0 + ((j + 1) * NBA + b) * KA, 8)
                    pltpu.async_copy(src_hbm.at[pl.ds(off, KA)], sidx[b],
                                     sisem[b])

                pltpu.make_async_copy(
                    dst_hbm.at[pl.ds(e0, KA)], didx[b], disem[b]).wait()
                pltpu.async_copy(rows[b], acc_sh.at[didx[b]], ssem[b], add=True)

                @pl.when(j < NG - 1)
                def _():
                    off = pl.multiple_of(e0 + ((j + 1) * NBA + b) * KA, 8)
                    pltpu.make_async_copy(
                        rows[b], acc_sh.at[didx[b]], ssem[b]).wait()
                    pltpu.async_copy(dst_hbm.at[pl.ds(off, KA)], didx[b],
                                     disem[b])
                    pltpu.make_async_copy(
                        src_hbm.at[pl.ds(e0, KA)], sidx[b], sisem[b]).wait()
                    pltpu.async_copy(g_hbm.at[sidx[b]], rows[b], gsem[b])
            return carry

        lax.fori_loop(0, NG, group, 0)
        for b in range(NBA):
            pltpu.make_async_copy(rows[b], acc_sh.at[didx[b]], ssem[b]).wait()
        plsc.subcore_barrier()
        pltpu.sync_copy(acc_sh.at[pl.ds(s * rpt, rpt)],
                        out_hbm.at[c, pl.ds(s * rpt, rpt)])

    return k(g, src_pad, dst_pad, zerosD)


def _tc_matmul(x_pad, W, NP, D):
    grid = (NP // BR,)

    def body(x_ref, w_ref, h_ref):
        h_ref[...] = jnp.dot(x_ref[...], w_ref[...],
                             preferred_element_type=jnp.float32)

    return pl.pallas_call(
        body,
        grid=grid,
        in_specs=[
            pl.BlockSpec((BR, D), lambda i: (i, 0)),
            pl.BlockSpec((D, D), lambda i: (0, 0)),
        ],
        out_specs=pl.BlockSpec((BR, D), lambda i: (i, 0)),
        out_shape=jax.ShapeDtypeStruct((NP, D), jnp.float32),
    )(x_pad, W)


def _tc_scale(deg_parts, h, N, NP, D):
    grid = (NP // BR,)

    def body(degp_ref, h_ref, g_ref, dinv_ref):
        i = pl.program_id(0)
        degsum = degp_ref[0] + degp_ref[1]
        deg = degsum[:, 0:1] + 1.0
        row = lax.broadcasted_iota(jnp.int32, (BR, 1), 0) + i * BR
        dinv = jnp.where(row < N, lax.rsqrt(deg), 0.0)
        g_ref[...] = h_ref[...] * dinv
        dinv_ref[...] = jnp.broadcast_to(dinv, (BR, 8))

    return pl.pallas_call(
        body,
        grid=grid,
        in_specs=[
            pl.BlockSpec((NC, BR, DEGW), lambda i: (0, i, 0)),
            pl.BlockSpec((BR, D), lambda i: (i, 0)),
        ],
        out_specs=[
            pl.BlockSpec((BR, D), lambda i: (i, 0)),
            pl.BlockSpec((BR, 8), lambda i: (i, 0)),
        ],
        out_shape=[
            jax.ShapeDtypeStruct((NP, D), jnp.float32),
            jax.ShapeDtypeStruct((NP, 8), jnp.float32),
        ],
    )(deg_parts, h)


def _tc_combine(parts, g, dinv8, b2d, NP, D):
    grid = (NP // BR,)

    def body(p_ref, g_ref, dinv_ref, b_ref, o_ref):
        ssum = p_ref[0] + p_ref[1] + g_ref[...]
        o_ref[...] = ssum * dinv_ref[:, 0:1] + b_ref[...]

    return pl.pallas_call(
        body,
        grid=grid,
        in_specs=[
            pl.BlockSpec((NC, BR, D), lambda i: (0, i, 0)),
            pl.BlockSpec((BR, D), lambda i: (i, 0)),
            pl.BlockSpec((BR, 8), lambda i: (i, 0)),
            pl.BlockSpec((1, D), lambda i: (0, 0)),
        ],
        out_specs=pl.BlockSpec((BR, D), lambda i: (i, 0)),
        out_shape=jax.ShapeDtypeStruct((NP, D), jnp.float32),
    )(parts, g, dinv8, b2d)


def kernel(x, edge_index, W, b):
    N, D_in = x.shape
    D = W.shape[1]
    E = edge_index.shape[1]

    NP = ((N + BR - 1) // BR) * BR
    EPWd = ((E + NW * NB * K - 1) // (NW * NB * K)) * (NB * K)
    CHd = EPWd // K
    UNIT = NS * NBA * KA
    units = (E + 2 * UNIT - 1) // (2 * UNIT) * 2
    u0 = max(2, min(units - 2, int(round(units * CORE0_FRAC))))
    EPWa0 = (u0 * UNIT) // NS
    EPWa1 = ((units - u0) * UNIT) // NS

    filler = jnp.full((1,), N, dtype=edge_index.dtype)
    dst_pad = jnp.concatenate(
        [edge_index[1], jnp.broadcast_to(filler, (EPWd * NW - E,))])
    EPa = NS * (EPWa0 + EPWa1)
    src_pada = jnp.concatenate(
        [edge_index[0], jnp.broadcast_to(filler, (EPa - E,))])
    dst_pada = jnp.concatenate(
        [edge_index[1], jnp.broadcast_to(filler, (EPa - E,))])
    x_pad = jnp.pad(x, ((0, NP - N), (0, 0)))

    zerosD = jnp.zeros((NP, D), jnp.float32)
    onesW = jnp.ones((K, DEGW), jnp.float32)

    h = _tc_matmul(x_pad, W, NP, D)
    deg_parts = _sc_degree(dst_pad, zerosD, onesW, NP, EPWd, CHd)
    g, dinv8 = _tc_scale(deg_parts, h, N, NP, D)
    parts = _sc_edge_aggregate(g, src_pada, dst_pada, zerosD, NP, D, EPWa0, EPWa1)
    out = _tc_combine(parts, g, dinv8, b.reshape(1, D), NP, D)
    return out[:N]


# speedup vs baseline: 1.0393x; 1.0393x over previous
import functools

import jax
import jax.numpy as jnp
from jax import lax
from jax.experimental import pallas as pl
from jax.experimental.pallas import tpu as pltpu
from jax.experimental.pallas import tpu_sc as plsc

NC = 2
NS = 16
NW = NC * NS
K = 128
NB = 4
NBA = 4
KA = 64
CORE0_FRAC = 1.0
BR = 512
DEGW = 128


def _sc_degree(dst_pad, zerosW, onesW, NP, EPW, CH):
    mesh = plsc.VectorSubcoreMesh(
        core_axis_name="c", subcore_axis_name="s", num_cores=NC, num_subcores=NS)
    rpt = NP // NS
    NG = CH // NB

    @functools.partial(
        pl.kernel,
        out_type=jax.ShapeDtypeStruct((NC, NP, DEGW), jnp.float32),
        mesh=mesh,
        scratch_types=(
            [pltpu.VMEM((K,), jnp.int32) for _ in range(NB)]
            + [pltpu.VMEM((K, DEGW), jnp.float32)]
            + [pltpu.SemaphoreType.DMA for _ in range(2 * NB)]
            + [pltpu.VMEM_SHARED((NP, DEGW), jnp.float32)]
        ),
    )
    def k(dst_hbm, zeros_hbm, ones_hbm, out_hbm, *refs):
        didx = refs[0:NB]
        ones_v = refs[NB]
        isem = refs[NB + 1:2 * NB + 1]
        ssem = refs[2 * NB + 1:3 * NB + 1]
        acc_sh = refs[3 * NB + 1]
        c = lax.axis_index("c")
        s = lax.axis_index("s")
        wid = s * NC + c
        pltpu.sync_copy(zeros_hbm.at[pl.ds(s * rpt, rpt)],
                        acc_sh.at[pl.ds(s * rpt, rpt)])
        pltpu.sync_copy(ones_hbm, ones_v)
        plsc.subcore_barrier()
        e0 = pl.multiple_of(wid * EPW, 8)

        for b in range(NB):
            off = pl.multiple_of(e0 + b * K, 8)
            pltpu.async_copy(dst_hbm.at[pl.ds(off, K)], didx[b], isem[b])

        def group(j, carry):
            for b in range(NB):
                pltpu.make_async_copy(
                    dst_hbm.at[pl.ds(e0, K)], didx[b], isem[b]).wait()
                pltpu.async_copy(ones_v, acc_sh.at[didx[b]], ssem[b], add=True)

                @pl.when(j < NG - 1)
                def _():
                    off = pl.multiple_of(e0 + ((j + 1) * NB + b) * K, 8)
                    pltpu.make_async_copy(
                        ones_v, acc_sh.at[didx[b]], ssem[b]).wait()
                    pltpu.async_copy(dst_hbm.at[pl.ds(off, K)], didx[b], isem[b])
            return carry

        lax.fori_loop(0, NG, group, 0)
        for b in range(NB):
            pltpu.make_async_copy(ones_v, acc_sh.at[didx[b]], ssem[b]).wait()
        plsc.subcore_barrier()
        pltpu.sync_copy(acc_sh.at[pl.ds(s * rpt, rpt)],
                        out_hbm.at[c, pl.ds(s * rpt, rpt)])

    return k(dst_pad, zerosW, onesW)


def _sc_edge_aggregate(g, src_pad, dst_pad, zerosD, NP, D, EPW0, EPW1):
    mesh = plsc.VectorSubcoreMesh(
        core_axis_name="c", subcore_axis_name="s", num_cores=NC, num_subcores=NS)
    rpt = NP // NS
    NG0 = EPW0 // (NBA * KA)
    NG1 = EPW1 // (NBA * KA)

    @functools.partial(
        pl.kernel,
        out_type=jax.ShapeDtypeStruct((NC, NP, D), jnp.float32),
        mesh=mesh,
        scratch_types=(
            [pltpu.VMEM((KA,), jnp.int32) for _ in range(2 * NBA)]
            + [pltpu.VMEM((KA, D), jnp.float32) for _ in range(NBA)]
            + [pltpu.SemaphoreType.DMA for _ in range(4 * NBA)]
            + [pltpu.VMEM_SHARED((NP, D), jnp.float32)]
        ),
    )
    def k(g_hbm, src_hbm, dst_hbm, zeros_hbm, out_hbm, *refs):
        sidx = refs[0:NBA]
        didx = refs[NBA:2 * NBA]
        rows = refs[2 * NBA:3 * NBA]
        sisem = refs[3 * NBA:4 * NBA]
        disem = refs[4 * NBA:5 * NBA]
        gsem = refs[5 * NBA:6 * NBA]
        ssem = refs[6 * NBA:7 * NBA]
        acc_sh = refs[7 * NBA]
        c = lax.axis_index("c")
        s = lax.axis_index("s")
        wid = s * NC + c
        pltpu.sync_copy(zeros_hbm.at[pl.ds(s * rpt, rpt)],
                        acc_sh.at[pl.ds(s * rpt, rpt)])
        plsc.subcore_barrier()
        epw = jnp.where(c == 0, EPW0, EPW1)
        NG = jnp.where(c == 0, NG0, NG1)
        e0 = pl.multiple_of(c * (NS * EPW0) + s * epw, 8)

        for b in range(NBA):
            off = pl.multiple_of(e0 + b * KA, 8)
            pltpu.async_copy(src_hbm.at[pl.ds(off, KA)], sidx[b], sisem[b])
            pltpu.async_copy(dst_hbm.at[pl.ds(off, KA)], didx[b], disem[b])
            pltpu.make_async_copy(
                src_hbm.at[pl.ds(e0, KA)], sidx[b], sisem[b]).wait()
            pltpu.async_copy(g_hbm.at[sidx[b]], rows[b], gsem[b])

        def group(j, carry):
            for b in range(NBA):
                pltpu.make_async_copy(
                    g_hbm.at[sidx[b]], rows[b], gsem[b]).wait()

                @pl.when(j < NG - 1)
                def _():
                    off = pl.multiple_of(e0 + ((j + 1) * NBA + b) * KA, 8)
                    pltpu.async_copy(src_hbm.at[pl.ds(off, KA)], sidx[b],
                                     sisem[b])

                pltpu.make_async_copy(
                    dst_hbm.at[pl.ds(e0, KA)], didx[b], disem[b]).wait()
                pltpu.async_copy(rows[b], acc_sh.at[didx[b]], ssem[b], add=True)

                @pl.when(j < NG - 1)
                def _():
                    off = pl.multiple_of(e0 + ((j + 1) * NBA + b) * KA, 8)
                    pltpu.make_async_copy(
                        rows[b], acc_sh.at[didx[b]], ssem[b]).wait()
                    pltpu.async_copy(dst_hbm.at[pl.ds(off, KA)], didx[b],
                                     disem[b])
                    pltpu.make_async_copy(
                        src_hbm.at[pl.ds(e0, KA)], sidx[b], sisem[b]).wait()
                    pltpu.async_copy(g_hbm.at[sidx[b]], rows[b], gsem[b])
            return carry

        lax.fori_loop(0, NG, group, 0)
        for b in range(NBA):
            pltpu.make_async_copy(rows[b], acc_sh.at[didx[b]], ssem[b]).wait()
        plsc.subcore_barrier()
        pltpu.sync_copy(acc_sh.at[pl.ds(s * rpt, rpt)],
                        out_hbm.at[c, pl.ds(s * rpt, rpt)])

    return k(g, src_pad, dst_pad, zerosD)


def _tc_transform(deg_parts, x_pad, W, N, NP, D):
    grid = (NP // BR,)

    def body(degp_ref, x_ref, w_ref, g_ref, dinv_ref):
        i = pl.program_id(0)
        degsum = degp_ref[0] + degp_ref[1]
        deg = degsum[:, 0:1] + 1.0
        row = lax.broadcasted_iota(jnp.int32, (BR, 1), 0) + i * BR
        dinv = jnp.where(row < N, lax.rsqrt(deg), 0.0)
        h = jnp.dot(x_ref[...], w_ref[...], preferred_element_type=jnp.float32)
        g_ref[...] = h * dinv
        dinv_ref[...] = jnp.broadcast_to(dinv, (BR, 8))

    return pl.pallas_call(
        body,
        grid=grid,
        in_specs=[
            pl.BlockSpec((NC, BR, DEGW), lambda i: (0, i, 0)),
            pl.BlockSpec((BR, D), lambda i: (i, 0)),
            pl.BlockSpec((D, D), lambda i: (0, 0)),
        ],
        out_specs=[
            pl.BlockSpec((BR, D), lambda i: (i, 0)),
            pl.BlockSpec((BR, 8), lambda i: (i, 0)),
        ],
        out_shape=[
            jax.ShapeDtypeStruct((NP, D), jnp.float32),
            jax.ShapeDtypeStruct((NP, 8), jnp.float32),
        ],
    )(deg_parts, x_pad, W)


def _tc_combine(parts, g, dinv8, b2d, NP, D):
    grid = (NP // BR,)

    def body(p_ref, g_ref, dinv_ref, b_ref, o_ref):
        ssum = p_ref[0] + p_ref[1] + g_ref[...]
        o_ref[...] = ssum * dinv_ref[:, 0:1] + b_ref[...]

    return pl.pallas_call(
        body,
        grid=grid,
        in_specs=[
            pl.BlockSpec((NC, BR, D), lambda i: (0, i, 0)),
            pl.BlockSpec((BR, D), lambda i: (i, 0)),
            pl.BlockSpec((BR, 8), lambda i: (i, 0)),
            pl.BlockSpec((1, D), lambda i: (0, 0)),
        ],
        out_specs=pl.BlockSpec((BR, D), lambda i: (i, 0)),
        out_shape=jax.ShapeDtypeStruct((NP, D), jnp.float32),
    )(parts, g, dinv8, b2d)


def kernel(x, edge_index, W, b):
    N, D_in = x.shape
    D = W.shape[1]
    E = edge_index.shape[1]

    NP = ((N + BR - 1) // BR) * BR
    EPWd = ((E + NW * NB * K - 1) // (NW * NB * K)) * (NB * K)
    CHd = EPWd // K
    UNIT = NS * NBA * KA
    units = (E + 2 * UNIT - 1) // (2 * UNIT) * 2
    u0 = max(2, min(units - 2, int(round(units * CORE0_FRAC))))
    EPWa0 = (u0 * UNIT) // NS
    EPWa1 = ((units - u0) * UNIT) // NS

    filler = jnp.full((1,), N, dtype=edge_index.dtype)
    dst_pad = jnp.concatenate(
        [edge_index[1], jnp.broadcast_to(filler, (EPWd * NW - E,))])
    EPa = NS * (EPWa0 + EPWa1)
    src_pada = jnp.concatenate(
        [edge_index[0], jnp.broadcast_to(filler, (EPa - E,))])
    dst_pada = jnp.concatenate(
        [edge_index[1], jnp.broadcast_to(filler, (EPa - E,))])
    x_pad = jnp.pad(x, ((0, NP - N), (0, 0)))

    zerosD = jnp.zeros((NP, D), jnp.float32)
    onesW = jnp.ones((K, DEGW), jnp.float32)

    deg_parts = _sc_degree(dst_pad, zerosD, onesW, NP, EPWd, CHd)
    g, dinv8 = _tc_transform(deg_parts, x_pad, W, N, NP, D)
    parts = _sc_edge_aggregate(g, src_pada, dst_pada, zerosD, NP, D, EPWa0, EPWa1)
    out = _tc_combine(parts, g, dinv8, b.reshape(1, D), NP, D)
    return out[:N]
